# baseline (speedup 1.0000x reference)
import functools

import jax
import jax.numpy as jnp
from jax import lax
from jax.experimental import pallas as pl
from jax.experimental.pallas import tpu as pltpu
from jax.experimental.pallas import tpu_sc as plsc

_NUM_CORES = 2
_NUM_SUBCORES = 16
_NW = _NUM_CORES * _NUM_SUBCORES

_CHUNK = 128
_NBUF = 8


@functools.lru_cache(maxsize=None)
def _make_sc_gather(n_rows: int, d: int):
  assert n_rows % (_NW * _CHUNK) == 0, n_rows
  nch = n_rows // (_NW * _CHUNK)
  nbuf = _NBUF
  while nch % nbuf:
    nbuf //= 2
  g_total = nch // nbuf

  mesh = plsc.VectorSubcoreMesh(core_axis_name="c", subcore_axis_name="s")

  @functools.partial(
      pl.kernel,
      mesh=mesh,
      out_type=jax.ShapeDtypeStruct((_NW * nch, _CHUNK, d), jnp.float32),
      compiler_params=pltpu.CompilerParams(use_tc_tiling_on_sc=False),
      scratch_types=(
          [pltpu.VMEM((nch, _CHUNK), jnp.int32)]
          + [pltpu.VMEM((_CHUNK, d), jnp.float32) for _ in range(nbuf)]
          + [pltpu.SemaphoreType.DMA for _ in range(2 * nbuf)]
      ),
  )
  def sc_gather(idx_hbm, table_hbm, out_hbm, idx_v, *rest):
    rbufs = rest[:nbuf]
    gsems = rest[nbuf:2 * nbuf]
    wsems = rest[2 * nbuf:]
    wid = lax.axis_index("s") * _NUM_CORES + lax.axis_index("c")

    pltpu.sync_copy(idx_hbm.at[wid], idx_v)

    def start_gather(b, j):
      pltpu.async_copy(table_hbm.at[idx_v.at[j]], rbufs[b], gsems[b])

    def wait_gather(b, j):
      pltpu.make_async_copy(
          table_hbm.at[idx_v.at[j]], rbufs[b], gsems[b]).wait()

    def start_write(b, j):
      pltpu.async_copy(rbufs[b], out_hbm.at[wid * nch + j], wsems[b])

    def wait_write(b, j):
      pltpu.make_async_copy(
          rbufs[b], out_hbm.at[wid * nch + j], wsems[b]).wait()

    for b in range(nbuf):
      start_gather(b, b)

    def body(g, carry):
      j0 = g * nbuf
      for b in range(nbuf):
        wait_gather(b, j0 + b)
        start_write(b, j0 + b)
      for b in range(nbuf):
        wait_write(b, j0 + b)
        start_gather(b, j0 + nbuf + b)
      return carry

    lax.fori_loop(0, g_total - 1, body, 0)

    j0 = (g_total - 1) * nbuf
    for b in range(nbuf):
      wait_gather(b, j0 + b)
      start_write(b, j0 + b)
    for b in range(nbuf):
      wait_write(b, j0 + b)

  return sc_gather


def kernel(indices, table):
  b, s = indices.shape
  v, d = table.shape
  n = b * s
  idx3 = indices.astype(jnp.int32).reshape(_NW, n // (_NW * _CHUNK), _CHUNK)
  out = _make_sc_gather(n, d)(idx3, table)
  return out.reshape(b, s, d)
